# baseline (speedup 1.0000x reference)
import jax
import jax.numpy as jnp
import numpy as np
from jax.experimental import pallas as pl
from jax.experimental.pallas import tpu as pltpu

_SIZE = 128
_LO = -0.0001
_HI = 1.0001
_STEP = (_HI - _LO) / _SIZE

_ROWS = 256
_NBLK = 4096 // _ROWS
_NBUF = 3
_HALF = _ROWS // 2


def _rbf_body(sg_ref, pts_ref, out_hbm, scratch, sems):
    k = pl.program_id(0)
    buf = jax.lax.rem(k, _NBUF)

    s = sg_ref[0]
    inv = -0.5 / (s * s)
    norm = 1.0 / (2.0 * np.float32(np.pi) * s * s)

    x = pts_ref[:, 0:1]
    p = pts_ref[:, 1:2] - x

    ci = _LO + _STEP * jax.lax.broadcasted_iota(
        jnp.int32, (1, _SIZE), 1).astype(jnp.float32)

    gxn = jnp.exp((x - ci) * (x - ci) * inv) * norm
    gy = jnp.exp((p - ci) * (p - ci) * inv)
    pb = jnp.broadcast_to(p, (_ROWS, _SIZE))

    @pl.when(k >= _NBUF)
    def _():
        for h in range(2):
            pltpu.make_async_copy(
                scratch.at[buf, pl.ds(h * _HALF, _HALF), :],
                out_hbm.at[pl.ds((k - _NBUF) * _ROWS + h * _HALF, _HALF), :],
                sems.at[buf, h],
            ).wait()

    view = scratch.at[buf]
    for j in range(_SIZE):
        sl = slice(_SIZE * j, _SIZE * (j + 1))
        if j % 2 == 0:
            view[:, sl] = gxn * gy[:, j:j + 1]
        else:
            t = pb - (_LO + _STEP * j)
            view[:, sl] = gxn * jnp.exp(t * t * inv)

    for h in range(2):
        pltpu.make_async_copy(
            scratch.at[buf, pl.ds(h * _HALF, _HALF), :],
            out_hbm.at[pl.ds(k * _ROWS + h * _HALF, _HALF), :],
            sems.at[buf, h],
        ).start()

    @pl.when(k == _NBLK - 1)
    def _():
        for off in range(_NBUF - 1, -1, -1):
            kk = _NBLK - 1 - off
            bb = jax.lax.rem(jnp.int32(kk), _NBUF)
            for h in range(2):
                pltpu.make_async_copy(
                    scratch.at[bb, pl.ds(h * _HALF, _HALF), :],
                    out_hbm.at[pl.ds(kk * _ROWS + h * _HALF, _HALF), :],
                    sems.at[bb, h],
                ).wait()


def kernel(inp, sg):
    B, P, _ = inp.shape
    n = B * P
    pts = inp.reshape(n, 2)
    out = pl.pallas_call(
        _rbf_body,
        out_shape=jax.ShapeDtypeStruct((n, _SIZE * _SIZE), jnp.float32),
        grid=(_NBLK,),
        in_specs=[
            pl.BlockSpec(memory_space=pltpu.SMEM),
            pl.BlockSpec((_ROWS, 2), lambda i: (i, 0)),
        ],
        out_specs=pl.BlockSpec(memory_space=pl.ANY),
        scratch_shapes=[
            pltpu.VMEM((_NBUF, _ROWS, _SIZE * _SIZE), jnp.float32),
            pltpu.SemaphoreType.DMA((_NBUF, 2)),
        ],
        compiler_params=pltpu.CompilerParams(
            dimension_semantics=("arbitrary",),
            vmem_limit_bytes=56 * 1024 * 1024,
        ),
        name="rbf_splat",
    )(sg, pts)
    return out.reshape(B, P, _SIZE * _SIZE)


# speedup vs baseline: 1.0011x; 1.0011x over previous
import jax
import jax.numpy as jnp
import numpy as np
from jax.experimental import pallas as pl
from jax.experimental.pallas import tpu as pltpu

_SIZE = 128
_LO = -0.0001
_HI = 1.0001
_STEP = (_HI - _LO) / _SIZE

_ROWS = 256
_NBLK = 4096 // _ROWS
_NBUF = 3


def _rbf_body(sg_ref, pts_ref, out_hbm, scratch, sems):
    k = pl.program_id(0)
    buf = jax.lax.rem(k, _NBUF)

    s = sg_ref[0]
    inv = -0.5 / (s * s)
    norm = 1.0 / (2.0 * np.float32(np.pi) * s * s)

    x = pts_ref[:, 0:1]
    p = pts_ref[:, 1:2] - x

    ci = _LO + _STEP * jax.lax.broadcasted_iota(
        jnp.int32, (1, _SIZE), 1).astype(jnp.float32)

    gxn = jnp.exp((x - ci) * (x - ci) * inv) * norm
    gy = jnp.exp((p - ci) * (p - ci) * inv)
    pb = jnp.broadcast_to(p, (_ROWS, _SIZE))

    @pl.when(k >= _NBUF)
    def _():
        pltpu.make_async_copy(
            scratch.at[buf],
            out_hbm.at[pl.ds((k - _NBUF) * _ROWS, _ROWS), :],
            sems.at[buf],
        ).wait()

    view = scratch.at[buf]
    for j in range(_SIZE):
        sl = slice(_SIZE * j, _SIZE * (j + 1))
        if j % 2 == 0:
            view[:, sl] = gxn * gy[:, j:j + 1]
        else:
            t = pb - (_LO + _STEP * j)
            view[:, sl] = gxn * jnp.exp(t * t * inv)

    pltpu.make_async_copy(
        scratch.at[buf],
        out_hbm.at[pl.ds(k * _ROWS, _ROWS), :],
        sems.at[buf],
    ).start()

    @pl.when(k == _NBLK - 1)
    def _():
        for off in range(_NBUF - 1, -1, -1):
            kk = _NBLK - 1 - off
            bb = jax.lax.rem(jnp.int32(kk), _NBUF)
            pltpu.make_async_copy(
                scratch.at[bb],
                out_hbm.at[pl.ds(kk * _ROWS, _ROWS), :],
                sems.at[bb],
            ).wait()


def kernel(inp, sg):
    B, P, _ = inp.shape
    n = B * P
    pts = inp.reshape(n, 2)
    out = pl.pallas_call(
        _rbf_body,
        out_shape=jax.ShapeDtypeStruct((n, _SIZE * _SIZE), jnp.float32),
        grid=(_NBLK,),
        in_specs=[
            pl.BlockSpec(memory_space=pltpu.SMEM),
            pl.BlockSpec((_ROWS, 2), lambda i: (i, 0)),
        ],
        out_specs=pl.BlockSpec(memory_space=pl.ANY),
        scratch_shapes=[
            pltpu.VMEM((_NBUF, _ROWS, _SIZE * _SIZE), jnp.float32),
            pltpu.SemaphoreType.DMA((_NBUF,)),
        ],
        compiler_params=pltpu.CompilerParams(
            dimension_semantics=("arbitrary",),
            vmem_limit_bytes=56 * 1024 * 1024,
        ),
        name="rbf_splat",
    )(sg, pts)
    return out.reshape(B, P, _SIZE * _SIZE)


# speedup vs baseline: 1.0068x; 1.0057x over previous
import jax
import jax.numpy as jnp
import numpy as np
from jax.experimental import pallas as pl
from jax.experimental.pallas import tpu as pltpu

_SIZE = 128
_LO = -0.0001
_HI = 1.0001
_STEP = (_HI - _LO) / _SIZE

_ROWS = 256
_NBLK = 4096 // _ROWS
_NBUF = 3
_HALF = _ROWS // 2


def _rbf_body(sg_ref, pts_ref, out_hbm, scratch, sems):
    k = pl.program_id(0)
    buf = jax.lax.rem(k, _NBUF)

    s = sg_ref[0]
    inv = -0.5 / (s * s)
    norm = 1.0 / (2.0 * np.float32(np.pi) * s * s)

    x = pts_ref[:, 0:1]
    p = pts_ref[:, 1:2] - x

    ci = _LO + _STEP * jax.lax.broadcasted_iota(
        jnp.int32, (1, _SIZE), 1).astype(jnp.float32)

    gxn = jnp.exp((x - ci) * (x - ci) * inv) * norm
    gy = jnp.exp((p - ci) * (p - ci) * inv)
    pb = jnp.broadcast_to(p, (_ROWS, _SIZE))

    @pl.when(k >= _NBUF)
    def _():
        for h in range(2):
            pltpu.make_async_copy(
                scratch.at[buf, pl.ds(h * _HALF, _HALF), :],
                out_hbm.at[pl.ds((k - _NBUF) * _ROWS + h * _HALF, _HALF), :],
                sems.at[buf, h],
            ).wait()

    view = scratch.at[buf]
    for h in range(2):
        rs = slice(h * _HALF, (h + 1) * _HALF)
        gxn_h = gxn[rs]
        gy_h = gy[rs]
        pb_h = pb[rs]
        for j in range(_SIZE):
            sl = slice(_SIZE * j, _SIZE * (j + 1))
            if j % 2 == 0:
                view[rs, sl] = gxn_h * gy_h[:, j:j + 1]
            else:
                t = pb_h - (_LO + _STEP * j)
                view[rs, sl] = gxn_h * jnp.exp(t * t * inv)
        pltpu.make_async_copy(
            scratch.at[buf, pl.ds(h * _HALF, _HALF), :],
            out_hbm.at[pl.ds(k * _ROWS + h * _HALF, _HALF), :],
            sems.at[buf, h],
        ).start()

    @pl.when(k == _NBLK - 1)
    def _():
        for off in range(_NBUF - 1, -1, -1):
            kk = _NBLK - 1 - off
            bb = jax.lax.rem(jnp.int32(kk), _NBUF)
            for h in range(2):
                pltpu.make_async_copy(
                    scratch.at[bb, pl.ds(h * _HALF, _HALF), :],
                    out_hbm.at[pl.ds(kk * _ROWS + h * _HALF, _HALF), :],
                    sems.at[bb, h],
                ).wait()


def kernel(inp, sg):
    B, P, _ = inp.shape
    n = B * P
    pts = inp.reshape(n, 2)
    out = pl.pallas_call(
        _rbf_body,
        out_shape=jax.ShapeDtypeStruct((n, _SIZE * _SIZE), jnp.float32),
        grid=(_NBLK,),
        in_specs=[
            pl.BlockSpec(memory_space=pltpu.SMEM),
            pl.BlockSpec((_ROWS, 2), lambda i: (i, 0)),
        ],
        out_specs=pl.BlockSpec(memory_space=pl.ANY),
        scratch_shapes=[
            pltpu.VMEM((_NBUF, _ROWS, _SIZE * _SIZE), jnp.float32),
            pltpu.SemaphoreType.DMA((_NBUF, 2)),
        ],
        compiler_params=pltpu.CompilerParams(
            dimension_semantics=("arbitrary",),
            vmem_limit_bytes=56 * 1024 * 1024,
        ),
        name="rbf_splat",
    )(sg, pts)
    return out.reshape(B, P, _SIZE * _SIZE)


# speedup vs baseline: 1.0134x; 1.0066x over previous
import jax
import jax.numpy as jnp
import numpy as np
from jax.experimental import pallas as pl
from jax.experimental.pallas import tpu as pltpu

_SIZE = 128
_LO = -0.0001
_HI = 1.0001
_STEP = (_HI - _LO) / _SIZE

_ROWS = 256
_NBLK = 4096 // _ROWS
_NBUF = 3
_HALF = _ROWS // 2


def _rbf_body(sg_ref, pts_ref, out_hbm, scratch, sems):
    k = pl.program_id(0)
    buf = jax.lax.rem(k, _NBUF)

    s = sg_ref[0]
    inv = -0.5 / (s * s)
    norm = 1.0 / (2.0 * np.float32(np.pi) * s * s)

    x = pts_ref[:, 0:1].astype(jnp.bfloat16).astype(jnp.float32)
    y = pts_ref[:, 1:2].astype(jnp.bfloat16).astype(jnp.float32)
    p = y - x

    ci = _LO + _STEP * jax.lax.broadcasted_iota(
        jnp.int32, (1, _SIZE), 1).astype(jnp.float32)

    _FLOOR = -87.0
    gxn = jnp.exp(jnp.maximum((x - ci) * (x - ci) * inv, _FLOOR)) * norm
    gy = jnp.exp(jnp.maximum((p - ci) * (p - ci) * inv, _FLOOR))
    pb = jnp.broadcast_to(p, (_ROWS, _SIZE))

    @pl.when(k >= _NBUF)
    def _():
        for h in range(2):
            pltpu.make_async_copy(
                scratch.at[buf, pl.ds(h * _HALF, _HALF), :],
                out_hbm.at[pl.ds((k - _NBUF) * _ROWS + h * _HALF, _HALF), :],
                sems.at[buf, h],
            ).wait()

    view = scratch.at[buf]
    for h in range(2):
        rs = slice(h * _HALF, (h + 1) * _HALF)
        gxn_h = gxn[rs]
        gy_h = gy[rs]
        pb_h = pb[rs]
        for j in range(_SIZE):
            sl = slice(_SIZE * j, _SIZE * (j + 1))
            if j % 2 == 0:
                view[rs, sl] = gxn_h * gy_h[:, j:j + 1]
            else:
                t = pb_h - (_LO + _STEP * j)
                view[rs, sl] = gxn_h * jnp.exp(
                    jnp.maximum(t * t * inv, _FLOOR))
        pltpu.make_async_copy(
            scratch.at[buf, pl.ds(h * _HALF, _HALF), :],
            out_hbm.at[pl.ds(k * _ROWS + h * _HALF, _HALF), :],
            sems.at[buf, h],
        ).start()

    @pl.when(k == _NBLK - 1)
    def _():
        for off in range(_NBUF - 1, -1, -1):
            kk = _NBLK - 1 - off
            bb = jax.lax.rem(jnp.int32(kk), _NBUF)
            for h in range(2):
                pltpu.make_async_copy(
                    scratch.at[bb, pl.ds(h * _HALF, _HALF), :],
                    out_hbm.at[pl.ds(kk * _ROWS + h * _HALF, _HALF), :],
                    sems.at[bb, h],
                ).wait()


def kernel(inp, sg):
    B, P, _ = inp.shape
    n = B * P
    pts = inp.reshape(n, 2)
    out = pl.pallas_call(
        _rbf_body,
        out_shape=jax.ShapeDtypeStruct((n, _SIZE * _SIZE), jnp.float32),
        grid=(_NBLK,),
        in_specs=[
            pl.BlockSpec(memory_space=pltpu.SMEM),
            pl.BlockSpec((_ROWS, 2), lambda i: (i, 0)),
        ],
        out_specs=pl.BlockSpec(memory_space=pl.ANY),
        scratch_shapes=[
            pltpu.VMEM((_NBUF, _ROWS, _SIZE * _SIZE), jnp.float32),
            pltpu.SemaphoreType.DMA((_NBUF, 2)),
        ],
        compiler_params=pltpu.CompilerParams(
            dimension_semantics=("arbitrary",),
            vmem_limit_bytes=56 * 1024 * 1024,
        ),
        name="rbf_splat",
    )(sg, pts)
    return out.reshape(B, P, _SIZE * _SIZE)


# speedup vs baseline: 1.0142x; 1.0007x over previous
import jax
import jax.numpy as jnp
import numpy as np
from jax.experimental import pallas as pl
from jax.experimental.pallas import tpu as pltpu

_SIZE = 128
_LO = -0.0001
_HI = 1.0001
_STEP = (_HI - _LO) / _SIZE

_ROWS = 256
_NBLK = 4096 // _ROWS
_NBUF = 3
_HALF = _ROWS // 2


def _rbf_body(sg_ref, pts_ref, out_hbm, scratch, sems):
    k = pl.program_id(0)
    buf = jax.lax.rem(k, _NBUF)

    s = sg_ref[0]
    inv = -0.5 / (s * s)
    norm = 1.0 / (2.0 * np.float32(np.pi) * s * s)

    x = pts_ref[:, 0:1].astype(jnp.bfloat16).astype(jnp.float32)
    y = pts_ref[:, 1:2].astype(jnp.bfloat16).astype(jnp.float32)
    p = y - x

    ci = _LO + _STEP * jax.lax.broadcasted_iota(
        jnp.int32, (1, _SIZE), 1).astype(jnp.float32)

    _FLOOR = -87.0
    gxn = jnp.exp(jnp.maximum((x - ci) * (x - ci) * inv, _FLOOR)) * norm
    gy = jnp.exp(jnp.maximum((p - ci) * (p - ci) * inv, _FLOOR))
    pb = jnp.broadcast_to(p, (_ROWS, _SIZE))

    @pl.when(k >= _NBUF)
    def _():
        for h in range(2):
            pltpu.make_async_copy(
                scratch.at[buf, pl.ds(h * _HALF, _HALF), :],
                out_hbm.at[pl.ds((k - _NBUF) * _ROWS + h * _HALF, _HALF), :],
                sems.at[buf, h],
            ).wait()

    view = scratch.at[buf]
    for h in range(2):
        rs = slice(h * _HALF, (h + 1) * _HALF)
        gxn_h = gxn[rs]
        gy_h = gy[rs]
        pb_h = pb[rs]
        for j in range(_SIZE):
            sl = slice(_SIZE * j, _SIZE * (j + 1))
            if j % 2 == 0:
                view[rs, sl] = gxn_h * gy_h[:, j:j + 1]
            else:
                cj = float(np.float32(_LO)
                           + np.float32(np.float32(_STEP) * np.float32(j)))
                t = pb_h - cj
                view[rs, sl] = gxn_h * jnp.exp(
                    jnp.maximum(t * t * inv, _FLOOR))
        pltpu.make_async_copy(
            scratch.at[buf, pl.ds(h * _HALF, _HALF), :],
            out_hbm.at[pl.ds(k * _ROWS + h * _HALF, _HALF), :],
            sems.at[buf, h],
        ).start()

    @pl.when(k == _NBLK - 1)
    def _():
        for off in range(_NBUF - 1, -1, -1):
            kk = _NBLK - 1 - off
            bb = jax.lax.rem(jnp.int32(kk), _NBUF)
            for h in range(2):
                pltpu.make_async_copy(
                    scratch.at[bb, pl.ds(h * _HALF, _HALF), :],
                    out_hbm.at[pl.ds(kk * _ROWS + h * _HALF, _HALF), :],
                    sems.at[bb, h],
                ).wait()


def kernel(inp, sg):
    B, P, _ = inp.shape
    n = B * P
    pts = inp.reshape(n, 2)
    out = pl.pallas_call(
        _rbf_body,
        out_shape=jax.ShapeDtypeStruct((n, _SIZE * _SIZE), jnp.float32),
        grid=(_NBLK,),
        in_specs=[
            pl.BlockSpec(memory_space=pltpu.SMEM),
            pl.BlockSpec((_ROWS, 2), lambda i: (i, 0)),
        ],
        out_specs=pl.BlockSpec(memory_space=pl.ANY),
        scratch_shapes=[
            pltpu.VMEM((_NBUF, _ROWS, _SIZE * _SIZE), jnp.float32),
            pltpu.SemaphoreType.DMA((_NBUF, 2)),
        ],
        compiler_params=pltpu.CompilerParams(
            dimension_semantics=("arbitrary",),
            vmem_limit_bytes=56 * 1024 * 1024,
        ),
        name="rbf_splat",
    )(sg, pts)
    return out.reshape(B, P, _SIZE * _SIZE)
